# baseline (speedup 1.0000x reference)
import functools

import jax
import jax.numpy as jnp
from jax import lax
from jax.experimental import pallas as pl
from jax.experimental.pallas import tpu as pltpu
from jax.experimental.pallas import tpu_sc as plsc

N_NODES = 100000
NC, NS = 2, 16
NW = NC * NS
K = 4
CHUNK = K * 128
ZROWS = 448
ROWS_PER_TILE = 14 * ZROWS
N_PAD = NS * ROWS_PER_TILE
F = 8

_mesh = plsc.VectorSubcoreMesh(
    core_axis_name="c", subcore_axis_name="s", num_cores=NC, num_subcores=NS)


def _make_sc_pass(n_chunks, with_cnt):
  out_type = [jax.ShapeDtypeStruct((NC, N_PAD, F), jnp.float32)]
  scratch = [
      pltpu.VMEM_SHARED((N_PAD, F), jnp.float32),
      pltpu.VMEM((K, 128), jnp.int32),
      pltpu.VMEM((K, 128), jnp.int32),
      pltpu.VMEM((K, 128), jnp.int32),
      pltpu.VMEM((K, 128), jnp.int32),
      pltpu.VMEM((CHUNK, F), jnp.float32),
      pltpu.VMEM((CHUNK, F), jnp.float32),
      pltpu.VMEM((ZROWS, F), jnp.float32),
      pltpu.SemaphoreType.DMA,
      pltpu.SemaphoreType.DMA,
  ]
  if with_cnt:
    out_type.append(jax.ShapeDtypeStruct((NC, N_PAD, 1), jnp.float32))
    scratch.append(pltpu.VMEM_SHARED((N_PAD, 1), jnp.float32))
    scratch.append(pltpu.VMEM((128, 1), jnp.float32))
    scratch.append(pltpu.VMEM((ZROWS, 1), jnp.float32))

  @functools.partial(
      pl.kernel,
      out_type=tuple(out_type),
      mesh=_mesh,
      compiler_params=pltpu.CompilerParams(use_tc_tiling_on_sc=False),
      scratch_types=scratch,
  )
  def sc_pass(*refs):
    if with_cnt:
      (src_hbm, dst_hbm, table_hbm, ones_hbm, acc_out, cnt_out, acc_sh,
       sidx0, didx0, sidx1, didx1, rows0, rows1, zbuf, sem_g, sem_s,
       cnt_sh, ones_v, zbuf1) = refs
    else:
      (src_hbm, dst_hbm, table_hbm, acc_out, acc_sh,
       sidx0, didx0, sidx1, didx1, rows0, rows1, zbuf, sem_g, sem_s) = refs
    c = lax.axis_index("c")
    s = lax.axis_index("s")

    zero16 = jnp.zeros((16,), jnp.float32)

    def zb(i, carry):
      zbuf[i, :] = zero16[:F]
      if with_cnt:
        zbuf1[i, :] = zero16[:1]
      return carry

    lax.fori_loop(0, ZROWS, zb, 0)
    if with_cnt:
      pltpu.sync_copy(ones_hbm, ones_v)

    base_row = s * ROWS_PER_TILE

    def zc(r, carry):
      pltpu.sync_copy(zbuf, acc_sh.at[pl.ds(base_row + r * ZROWS, ZROWS)])
      if with_cnt:
        pltpu.sync_copy(zbuf1, cnt_sh.at[pl.ds(base_row + r * ZROWS, ZROWS)])
      return carry

    lax.fori_loop(0, ROWS_PER_TILE // ZROWS, zc, 0)
    plsc.subcore_barrier()

    w = c * NS + s

    def load_idx(g, sidx, didx):
      row0 = (w * n_chunks + g) * K
      pltpu.sync_copy(src_hbm.at[pl.ds(row0, K)], sidx)
      pltpu.sync_copy(dst_hbm.at[pl.ds(row0, K)], didx)

    def fire_gathers(sidx, rows):
      return [
          pltpu.async_copy(table_hbm.at[sidx.at[j]],
                           rows.at[pl.ds(j * 128, 128)], sem_g)
          for j in range(K)
      ]

    def fire_scatters(didx, rows):
      cps = [
          pltpu.async_copy(rows.at[pl.ds(j * 128, 128)],
                           acc_sh.at[didx.at[j]], sem_s, add=True)
          for j in range(K)
      ]
      if with_cnt:
        cps += [
            pltpu.async_copy(ones_v, cnt_sh.at[didx.at[j]], sem_s, add=True)
            for j in range(K)
        ]
      return cps

    def pair(i, carry):
      load_idx(2 * i, sidx0, didx0)
      ga = fire_gathers(sidx0, rows0)
      load_idx(2 * i + 1, sidx1, didx1)
      for cp in ga:
        cp.wait()
      sa = fire_scatters(didx0, rows0)
      gb = fire_gathers(sidx1, rows1)
      for cp in gb:
        cp.wait()
      sb = fire_scatters(didx1, rows1)
      for cp in sa:
        cp.wait()
      for cp in sb:
        cp.wait()
      return carry

    lax.fori_loop(0, n_chunks // 2, pair, 0)
    plsc.subcore_barrier()

    pltpu.sync_copy(acc_sh.at[pl.ds(base_row, ROWS_PER_TILE)],
                    acc_out.at[c, pl.ds(base_row, ROWS_PER_TILE)])
    if with_cnt:
      pltpu.sync_copy(cnt_sh.at[pl.ds(base_row, ROWS_PER_TILE)],
                      cnt_out.at[c, pl.ds(base_row, ROWS_PER_TILE)])

  return sc_pass


_BLK1 = 2048


def _tc_mean(acc, cnt):
  def body(acc_ref, cnt_ref, out_ref):
    a = acc_ref[0] + acc_ref[1]
    c = cnt_ref[0] + cnt_ref[1]
    out_ref[...] = a / jnp.maximum(c, 1.0)

  return pl.pallas_call(
      body,
      grid=(N_PAD // _BLK1,),
      in_specs=[
          pl.BlockSpec((NC, _BLK1, F), lambda i: (0, i, 0)),
          pl.BlockSpec((NC, _BLK1, 1), lambda i: (0, i, 0)),
      ],
      out_specs=pl.BlockSpec((_BLK1, F), lambda i: (i, 0)),
      out_shape=jax.ShapeDtypeStruct((N_PAD, F), jnp.float32),
  )(acc, cnt)


_BLK2 = 2000


def _tc_final(x, m1, acc2, cnt, w1l, w1r, b1, w2l, w2r, b2):
  def body(x_ref, m1_ref, acc_ref, cnt_ref, w1l_ref, w1r_ref, b1_ref,
           w2l_ref, w2r_ref, b2_ref, out_ref):
    c = cnt_ref[0] + cnt_ref[1]
    m2 = (acc_ref[0] + acc_ref[1]) / jnp.maximum(c, 1.0)
    mask = (c > 0.0).astype(jnp.float32)
    m1b = m1_ref[...]
    dot = functools.partial(jnp.dot, preferred_element_type=jnp.float32)
    h = dot(m1b, w1l_ref[...]) + b1_ref[...] + dot(x_ref[...], w1r_ref[...])
    ah = dot(m2, w1l_ref[...]) + mask * b1_ref[...] + dot(m1b, w1r_ref[...])
    out_ref[...] = dot(ah, w2l_ref[...]) + b2_ref[...] + dot(h, w2r_ref[...])

  return pl.pallas_call(
      body,
      grid=(N_NODES // _BLK2,),
      in_specs=[
          pl.BlockSpec((_BLK2, 8), lambda i: (i, 0)),
          pl.BlockSpec((_BLK2, F), lambda i: (i, 0)),
          pl.BlockSpec((NC, _BLK2, F), lambda i: (0, i, 0)),
          pl.BlockSpec((NC, _BLK2, 1), lambda i: (0, i, 0)),
          pl.BlockSpec((8, 16), lambda i: (0, 0)),
          pl.BlockSpec((8, 16), lambda i: (0, 0)),
          pl.BlockSpec((1, 16), lambda i: (0, 0)),
          pl.BlockSpec((16, 8), lambda i: (0, 0)),
          pl.BlockSpec((16, 8), lambda i: (0, 0)),
          pl.BlockSpec((1, 8), lambda i: (0, 0)),
      ],
      out_specs=pl.BlockSpec((_BLK2, 8), lambda i: (i, 0)),
      out_shape=jax.ShapeDtypeStruct((N_NODES, 8), jnp.float32),
  )(x, m1, acc2, cnt, w1l, w1r, b1, w2l, w2r, b2)


def kernel(x, edge_index, W1_l, W1_r, b1, W2_l, W2_r, b2):
  ei = edge_index.astype(jnp.int32)
  src, dst = ei[0], ei[1]
  e = src.shape[0]
  n_chunks = 2 * -(-e // (NW * CHUNK * 2))
  e_pad = NW * CHUNK * n_chunks
  pad = e_pad - e
  ar = jnp.arange(pad, dtype=jnp.int32)
  src_p = jnp.concatenate([src, ar % N_NODES]).reshape(e_pad // 128, 128)
  dst_p = jnp.concatenate(
      [dst, N_NODES + ar % (N_PAD - N_NODES)]).reshape(e_pad // 128, 128)

  ones = jnp.ones((128, 1), jnp.float32)
  acc1, cnt = _make_sc_pass(n_chunks, True)(src_p, dst_p, x, ones)
  m1 = _tc_mean(acc1, cnt)
  (acc2,) = _make_sc_pass(n_chunks, False)(src_p, dst_p, m1)

  return _tc_final(x, m1, acc2, cnt, W1_l, W1_r, b1.reshape(1, 16), W2_l,
                   W2_r, b2.reshape(1, 8))


# speedup vs baseline: 1.0002x; 1.0002x over previous
import functools

import jax
import jax.numpy as jnp
from jax import lax
from jax.experimental import pallas as pl
from jax.experimental.pallas import tpu as pltpu
from jax.experimental.pallas import tpu_sc as plsc

N_NODES = 100000
NC, NS = 2, 16
NW = NC * NS
K = 4
CHUNK = K * 128
ZROWS = 448
ROWS_PER_TILE = 14 * ZROWS
N_PAD = NS * ROWS_PER_TILE
F = 8

_mesh = plsc.VectorSubcoreMesh(
    core_axis_name="c", subcore_axis_name="s", num_cores=NC, num_subcores=NS)


def _make_sc_pass(n_chunks, with_cnt):
  out_type = [jax.ShapeDtypeStruct((NC, N_PAD, F), jnp.float32)]
  scratch = [
      pltpu.VMEM_SHARED((N_PAD, F), jnp.float32),
      pltpu.VMEM((K, 128), jnp.int32),
      pltpu.VMEM((K, 128), jnp.int32),
      pltpu.VMEM((K, 128), jnp.int32),
      pltpu.VMEM((K, 128), jnp.int32),
      pltpu.VMEM((CHUNK, F), jnp.float32),
      pltpu.VMEM((CHUNK, F), jnp.float32),
      pltpu.VMEM((ZROWS, F), jnp.float32),
      pltpu.SemaphoreType.DMA,
      pltpu.SemaphoreType.DMA,
  ]
  if with_cnt:
    out_type.append(jax.ShapeDtypeStruct((NC, N_PAD, F), jnp.float32))
    scratch.append(pltpu.VMEM_SHARED((N_PAD, F), jnp.float32))
    scratch.append(pltpu.VMEM((128, F), jnp.float32))

  @functools.partial(
      pl.kernel,
      out_type=tuple(out_type),
      mesh=_mesh,
      compiler_params=pltpu.CompilerParams(use_tc_tiling_on_sc=False),
      scratch_types=scratch,
  )
  def sc_pass(*refs):
    if with_cnt:
      (src_hbm, dst_hbm, table_hbm, ones_hbm, acc_out, cnt_out, acc_sh,
       sidx0, didx0, sidx1, didx1, rows0, rows1, zbuf, sem_g, sem_s,
       cnt_sh, ones_v) = refs
    else:
      (src_hbm, dst_hbm, table_hbm, acc_out, acc_sh,
       sidx0, didx0, sidx1, didx1, rows0, rows1, zbuf, sem_g, sem_s) = refs
    c = lax.axis_index("c")
    s = lax.axis_index("s")

    zero16 = jnp.zeros((16,), jnp.float32)

    def zb(i, carry):
      zbuf[i, :] = zero16[:F]
      return carry

    lax.fori_loop(0, ZROWS, zb, 0)
    if with_cnt:
      pltpu.sync_copy(ones_hbm, ones_v)

    base_row = s * ROWS_PER_TILE

    def zc(r, carry):
      pltpu.sync_copy(zbuf, acc_sh.at[pl.ds(base_row + r * ZROWS, ZROWS)])
      if with_cnt:
        pltpu.sync_copy(zbuf, cnt_sh.at[pl.ds(base_row + r * ZROWS, ZROWS)])
      return carry

    lax.fori_loop(0, ROWS_PER_TILE // ZROWS, zc, 0)
    plsc.subcore_barrier()

    w = c * NS + s

    def load_idx(g, sidx, didx):
      row0 = (w * n_chunks + g) * K
      pltpu.sync_copy(src_hbm.at[pl.ds(row0, K)], sidx)
      pltpu.sync_copy(dst_hbm.at[pl.ds(row0, K)], didx)

    def fire_gathers(sidx, rows):
      return [
          pltpu.async_copy(table_hbm.at[sidx.at[j]],
                           rows.at[pl.ds(j * 128, 128)], sem_g)
          for j in range(K)
      ]

    def fire_scatters(didx, rows):
      cps = [
          pltpu.async_copy(rows.at[pl.ds(j * 128, 128)],
                           acc_sh.at[didx.at[j]], sem_s, add=True)
          for j in range(K)
      ]
      if with_cnt:
        cps += [
            pltpu.async_copy(ones_v, cnt_sh.at[didx.at[j]], sem_s, add=True)
            for j in range(K)
        ]
      return cps

    def pair(i, carry):
      load_idx(2 * i, sidx0, didx0)
      ga = fire_gathers(sidx0, rows0)
      load_idx(2 * i + 1, sidx1, didx1)
      for cp in ga:
        cp.wait()
      sa = fire_scatters(didx0, rows0)
      gb = fire_gathers(sidx1, rows1)
      for cp in gb:
        cp.wait()
      sb = fire_scatters(didx1, rows1)
      for cp in sa:
        cp.wait()
      for cp in sb:
        cp.wait()
      return carry

    lax.fori_loop(0, n_chunks // 2, pair, 0)
    plsc.subcore_barrier()

    pltpu.sync_copy(acc_sh.at[pl.ds(base_row, ROWS_PER_TILE)],
                    acc_out.at[c, pl.ds(base_row, ROWS_PER_TILE)])
    if with_cnt:
      pltpu.sync_copy(cnt_sh.at[pl.ds(base_row, ROWS_PER_TILE)],
                      cnt_out.at[c, pl.ds(base_row, ROWS_PER_TILE)])

  return sc_pass


_BLK1 = 2048


def _tc_mean(acc, cnt):
  def body(acc_ref, cnt_ref, out_ref):
    a = acc_ref[0] + acc_ref[1]
    c = cnt_ref[0] + cnt_ref[1]
    out_ref[...] = a / jnp.maximum(c, 1.0)

  return pl.pallas_call(
      body,
      grid=(N_PAD // _BLK1,),
      in_specs=[
          pl.BlockSpec((NC, _BLK1, F), lambda i: (0, i, 0)),
          pl.BlockSpec((NC, _BLK1, F), lambda i: (0, i, 0)),
      ],
      out_specs=pl.BlockSpec((_BLK1, F), lambda i: (i, 0)),
      out_shape=jax.ShapeDtypeStruct((N_PAD, F), jnp.float32),
  )(acc, cnt)


_BLK2 = 2000


def _tc_final(x, m1, acc2, cnt, w1l, w1r, b1, w2l, w2r, b2):
  def body(x_ref, m1_ref, acc_ref, cnt_ref, w1l_ref, w1r_ref, b1_ref,
           w2l_ref, w2r_ref, b2_ref, out_ref):
    c = cnt_ref[0] + cnt_ref[1]
    m2 = (acc_ref[0] + acc_ref[1]) / jnp.maximum(c, 1.0)
    mask = (c[:, 0:1] > 0.0).astype(jnp.float32)
    m1b = m1_ref[...]
    dot = functools.partial(jnp.dot, preferred_element_type=jnp.float32)
    h = dot(m1b, w1l_ref[...]) + b1_ref[...] + dot(x_ref[...], w1r_ref[...])
    ah = dot(m2, w1l_ref[...]) + mask * b1_ref[...] + dot(m1b, w1r_ref[...])
    out_ref[...] = dot(ah, w2l_ref[...]) + b2_ref[...] + dot(h, w2r_ref[...])

  return pl.pallas_call(
      body,
      grid=(N_NODES // _BLK2,),
      in_specs=[
          pl.BlockSpec((_BLK2, 8), lambda i: (i, 0)),
          pl.BlockSpec((_BLK2, F), lambda i: (i, 0)),
          pl.BlockSpec((NC, _BLK2, F), lambda i: (0, i, 0)),
          pl.BlockSpec((NC, _BLK2, F), lambda i: (0, i, 0)),
          pl.BlockSpec((8, 16), lambda i: (0, 0)),
          pl.BlockSpec((8, 16), lambda i: (0, 0)),
          pl.BlockSpec((1, 16), lambda i: (0, 0)),
          pl.BlockSpec((16, 8), lambda i: (0, 0)),
          pl.BlockSpec((16, 8), lambda i: (0, 0)),
          pl.BlockSpec((1, 8), lambda i: (0, 0)),
      ],
      out_specs=pl.BlockSpec((_BLK2, 8), lambda i: (i, 0)),
      out_shape=jax.ShapeDtypeStruct((N_NODES, 8), jnp.float32),
  )(x, m1, acc2, cnt, w1l, w1r, b1, w2l, w2r, b2)


def kernel(x, edge_index, W1_l, W1_r, b1, W2_l, W2_r, b2):
  ei = edge_index.astype(jnp.int32)
  src, dst = ei[0], ei[1]
  e = src.shape[0]
  n_chunks = 2 * -(-e // (NW * CHUNK * 2))
  e_pad = NW * CHUNK * n_chunks
  pad = e_pad - e
  ar = jnp.arange(pad, dtype=jnp.int32)
  src_p = jnp.concatenate([src, ar % N_NODES]).reshape(e_pad // 128, 128)
  dst_p = jnp.concatenate(
      [dst, N_NODES + ar % (N_PAD - N_NODES)]).reshape(e_pad // 128, 128)

  ones = jnp.ones((128, F), jnp.float32)
  acc1, cnt = _make_sc_pass(n_chunks, True)(src_p, dst_p, x, ones)
  m1 = _tc_mean(acc1, cnt)
  (acc2,) = _make_sc_pass(n_chunks, False)(src_p, dst_p, m1)

  return _tc_final(x, m1, acc2, cnt, W1_l, W1_r, b1.reshape(1, 16), W2_l,
                   W2_r, b2.reshape(1, 8))


# speedup vs baseline: 1.1542x; 1.1540x over previous
import functools

import jax
import jax.numpy as jnp
from jax import lax
from jax.experimental import pallas as pl
from jax.experimental.pallas import tpu as pltpu
from jax.experimental.pallas import tpu_sc as plsc

N_NODES = 100000
NC, NS = 2, 16
NW = NC * NS
K = 4
CHUNK = K * 128
ZROWS = 448
ROWS_PER_TILE = 14 * ZROWS
N_PAD = NS * ROWS_PER_TILE
FA = 16
F = 8

_mesh = plsc.VectorSubcoreMesh(
    core_axis_name="c", subcore_axis_name="s", num_cores=NC, num_subcores=NS)

_sc_params = pltpu.CompilerParams(use_tc_tiling_on_sc=False)


def _edge_loop(n_chunks, w, load_idx, fire_gathers, fire_scatters, bufs):
  sidx0, didx0, sidx1, didx1, rows0, rows1 = bufs

  def pair(i, carry):
    load_idx(2 * i, sidx0, didx0)
    ga = fire_gathers(sidx0, rows0)
    load_idx(2 * i + 1, sidx1, didx1)
    for cp in ga:
      cp.wait()
    sa = fire_scatters(didx0, rows0)
    gb = fire_gathers(sidx1, rows1)
    for cp in gb:
      cp.wait()
    sb = fire_scatters(didx1, rows1)
    for cp in sa:
      cp.wait()
    for cp in sb:
      cp.wait()
    return carry

  lax.fori_loop(0, n_chunks // 2, pair, 0)


def _make_pass1(n_chunks):
  @functools.partial(
      pl.kernel,
      out_type=jax.ShapeDtypeStruct((NC, N_PAD, FA), jnp.float32),
      mesh=_mesh,
      compiler_params=_sc_params,
      scratch_types=[
          pltpu.VMEM_SHARED((N_PAD, FA), jnp.float32),
          pltpu.VMEM((K, 128), jnp.int32),
          pltpu.VMEM((K, 128), jnp.int32),
          pltpu.VMEM((K, 128), jnp.int32),
          pltpu.VMEM((K, 128), jnp.int32),
          pltpu.VMEM((CHUNK, FA), jnp.float32),
          pltpu.VMEM((CHUNK, FA), jnp.float32),
          pltpu.VMEM((ZROWS, FA), jnp.float32),
          pltpu.SemaphoreType.DMA,
          pltpu.SemaphoreType.DMA,
      ],
  )
  def sc_pass(src_hbm, dst_hbm, table_hbm, acc_out, acc_sh, sidx0, didx0,
              sidx1, didx1, rows0, rows1, zbuf, sem_g, sem_s):
    c = lax.axis_index("c")
    s = lax.axis_index("s")

    def zb(i, carry):
      zbuf[i, :] = jnp.zeros((16,), jnp.float32)
      return carry

    lax.fori_loop(0, ZROWS, zb, 0)

    base_row = s * ROWS_PER_TILE

    def zc(r, carry):
      pltpu.sync_copy(zbuf, acc_sh.at[pl.ds(base_row + r * ZROWS, ZROWS)])
      return carry

    lax.fori_loop(0, ROWS_PER_TILE // ZROWS, zc, 0)
    plsc.subcore_barrier()

    w = c * NS + s

    def load_idx(g, sidx, didx):
      row0 = (w * n_chunks + g) * K
      pltpu.sync_copy(src_hbm.at[pl.ds(row0, K)], sidx)
      pltpu.sync_copy(dst_hbm.at[pl.ds(row0, K)], didx)

    def fire_gathers(sidx, rows):
      return [
          pltpu.async_copy(table_hbm.at[sidx.at[j]],
                           rows.at[pl.ds(j * 128, 128)], sem_g)
          for j in range(K)
      ]

    def fire_scatters(didx, rows):
      return [
          pltpu.async_copy(rows.at[pl.ds(j * 128, 128)],
                           acc_sh.at[didx.at[j]], sem_s, add=True)
          for j in range(K)
      ]

    _edge_loop(n_chunks, w, load_idx, fire_gathers, fire_scatters,
               (sidx0, didx0, sidx1, didx1, rows0, rows1))
    plsc.subcore_barrier()

    pltpu.sync_copy(acc_sh.at[pl.ds(base_row, ROWS_PER_TILE)],
                    acc_out.at[c, pl.ds(base_row, ROWS_PER_TILE)])

  return sc_pass


def _make_pass2(n_chunks):
  @functools.partial(
      pl.kernel,
      out_type=jax.ShapeDtypeStruct((NC, N_PAD, F), jnp.float32),
      mesh=_mesh,
      compiler_params=_sc_params,
      scratch_types=[
          pltpu.VMEM_SHARED((N_PAD, F), jnp.float32),
          pltpu.VMEM_SHARED((N_PAD, F), jnp.float32),
          pltpu.VMEM((K, 128), jnp.int32),
          pltpu.VMEM((K, 128), jnp.int32),
          pltpu.VMEM((K, 128), jnp.int32),
          pltpu.VMEM((K, 128), jnp.int32),
          pltpu.VMEM((CHUNK, F), jnp.float32),
          pltpu.VMEM((CHUNK, F), jnp.float32),
          pltpu.VMEM((ZROWS, F), jnp.float32),
          pltpu.SemaphoreType.DMA,
          pltpu.SemaphoreType.DMA,
      ],
  )
  def sc_pass(src_hbm, dst_hbm, table_hbm, acc_out, acc_sh, table_sp,
              sidx0, didx0, sidx1, didx1, rows0, rows1, zbuf, sem_g, sem_s):
    c = lax.axis_index("c")
    s = lax.axis_index("s")

    def zb(i, carry):
      zbuf[i, :] = jnp.zeros((16,), jnp.float32)[:F]
      return carry

    lax.fori_loop(0, ZROWS, zb, 0)

    base_row = s * ROWS_PER_TILE
    def stage(r, carry):
      row = base_row + r * ZROWS
      pltpu.sync_copy(table_hbm.at[pl.ds(row, ZROWS)], rows0.at[pl.ds(0, ZROWS)])
      pltpu.sync_copy(rows0.at[pl.ds(0, ZROWS)], table_sp.at[pl.ds(row, ZROWS)])
      pltpu.sync_copy(zbuf, acc_sh.at[pl.ds(row, ZROWS)])
      return carry

    lax.fori_loop(0, ROWS_PER_TILE // ZROWS, stage, 0)
    plsc.subcore_barrier()

    w = c * NS + s

    def load_idx(g, sidx, didx):
      row0 = (w * n_chunks + g) * K
      pltpu.sync_copy(src_hbm.at[pl.ds(row0, K)], sidx)
      pltpu.sync_copy(dst_hbm.at[pl.ds(row0, K)], didx)

    def fire_gathers(sidx, rows):
      return [
          pltpu.async_copy(table_sp.at[sidx.at[j]],
                           rows.at[pl.ds(j * 128, 128)], sem_g)
          for j in range(K)
      ]

    def fire_scatters(didx, rows):
      return [
          pltpu.async_copy(rows.at[pl.ds(j * 128, 128)],
                           acc_sh.at[didx.at[j]], sem_s, add=True)
          for j in range(K)
      ]

    _edge_loop(n_chunks, w, load_idx, fire_gathers, fire_scatters,
               (sidx0, didx0, sidx1, didx1, rows0, rows1))
    plsc.subcore_barrier()

    pltpu.sync_copy(acc_sh.at[pl.ds(base_row, ROWS_PER_TILE)],
                    acc_out.at[c, pl.ds(base_row, ROWS_PER_TILE)])

  return sc_pass


_BLK1 = 2048


def _tc_mean(acc):
  def body(acc_ref, out_ref):
    a = acc_ref[0] + acc_ref[1]
    cnt = a[:, 8:9]
    out_ref[...] = a[:, 0:8] / jnp.maximum(cnt, 1.0)

  return pl.pallas_call(
      body,
      grid=(N_PAD // _BLK1,),
      in_specs=[pl.BlockSpec((NC, _BLK1, FA), lambda i: (0, i, 0))],
      out_specs=pl.BlockSpec((_BLK1, F), lambda i: (i, 0)),
      out_shape=jax.ShapeDtypeStruct((N_PAD, F), jnp.float32),
  )(acc)


_BLK2 = 2000


def _tc_final(x, m1, acc1, acc2, w1l, w1r, b1, w2l, w2r, b2):
  def body(x_ref, m1_ref, acc1_ref, acc2_ref, w1l_ref, w1r_ref, b1_ref,
           w2l_ref, w2r_ref, b2_ref, out_ref):
    cnt = (acc1_ref[0] + acc1_ref[1])[:, 8:9]
    m2 = (acc2_ref[0] + acc2_ref[1]) / jnp.maximum(cnt, 1.0)
    mask = (cnt > 0.0).astype(jnp.float32)
    m1b = m1_ref[...]
    dot = functools.partial(jnp.dot, preferred_element_type=jnp.float32)
    h = dot(m1b, w1l_ref[...]) + b1_ref[...] + dot(x_ref[...], w1r_ref[...])
    ah = dot(m2, w1l_ref[...]) + mask * b1_ref[...] + dot(m1b, w1r_ref[...])
    out_ref[...] = dot(ah, w2l_ref[...]) + b2_ref[...] + dot(h, w2r_ref[...])

  return pl.pallas_call(
      body,
      grid=(N_NODES // _BLK2,),
      in_specs=[
          pl.BlockSpec((_BLK2, 8), lambda i: (i, 0)),
          pl.BlockSpec((_BLK2, F), lambda i: (i, 0)),
          pl.BlockSpec((NC, _BLK2, FA), lambda i: (0, i, 0)),
          pl.BlockSpec((NC, _BLK2, F), lambda i: (0, i, 0)),
          pl.BlockSpec((8, 16), lambda i: (0, 0)),
          pl.BlockSpec((8, 16), lambda i: (0, 0)),
          pl.BlockSpec((1, 16), lambda i: (0, 0)),
          pl.BlockSpec((16, 8), lambda i: (0, 0)),
          pl.BlockSpec((16, 8), lambda i: (0, 0)),
          pl.BlockSpec((1, 8), lambda i: (0, 0)),
      ],
      out_specs=pl.BlockSpec((_BLK2, 8), lambda i: (i, 0)),
      out_shape=jax.ShapeDtypeStruct((N_NODES, 8), jnp.float32),
  )(x, m1, acc1, acc2, w1l, w1r, b1, w2l, w2r, b2)


def kernel(x, edge_index, W1_l, W1_r, b1, W2_l, W2_r, b2):
  ei = edge_index.astype(jnp.int32)
  src, dst = ei[0], ei[1]
  e = src.shape[0]
  n_chunks = 2 * -(-e // (NW * CHUNK * 2))
  e_pad = NW * CHUNK * n_chunks
  pad = e_pad - e
  ar = jnp.arange(pad, dtype=jnp.int32)
  src_p = jnp.concatenate([src, ar % N_NODES]).reshape(e_pad // 128, 128)
  dst_p = jnp.concatenate(
      [dst, N_NODES + ar % (N_PAD - N_NODES)]).reshape(e_pad // 128, 128)

  n = x.shape[0]
  x_aug = jnp.concatenate(
      [x, jnp.ones((n, 1), x.dtype), jnp.zeros((n, 7), x.dtype)], axis=1)

  acc1 = _make_pass1(n_chunks)(src_p, dst_p, x_aug)
  m1 = _tc_mean(acc1)
  acc2 = _make_pass2(n_chunks)(src_p, dst_p, m1)

  return _tc_final(x, m1, acc1, acc2, W1_l, W1_r, b1.reshape(1, 16), W2_l,
                   W2_r, b2.reshape(1, 8))


# speedup vs baseline: 1.3316x; 1.1537x over previous
import functools

import jax
import jax.numpy as jnp
from jax import lax
from jax.experimental import pallas as pl
from jax.experimental.pallas import tpu as pltpu
from jax.experimental.pallas import tpu_sc as plsc

N_NODES = 100000
NC, NS = 2, 16
NW = NC * NS
K = 6
CHUNK = K * 128
ZROWS = 112
ROWS_PER_TILE = 56 * ZROWS
N_PAD = NS * ROWS_PER_TILE
FA = 16
F = 8

_mesh = plsc.VectorSubcoreMesh(
    core_axis_name="c", subcore_axis_name="s", num_cores=NC, num_subcores=NS)

_sc_params = pltpu.CompilerParams(use_tc_tiling_on_sc=False)


def _edge_loop(n_chunks, w, load_idx, fire_gathers, fire_scatters, bufs):
  sidx0, didx0, sidx1, didx1, rows0, rows1 = bufs

  def pair(i, carry):
    load_idx(2 * i, sidx0, didx0)
    ga = fire_gathers(sidx0, rows0)
    load_idx(2 * i + 1, sidx1, didx1)
    for cp in ga:
      cp.wait()
    sa = fire_scatters(didx0, rows0)
    gb = fire_gathers(sidx1, rows1)
    for cp in gb:
      cp.wait()
    sb = fire_scatters(didx1, rows1)
    for cp in sa:
      cp.wait()
    for cp in sb:
      cp.wait()
    return carry

  lax.fori_loop(0, n_chunks // 2, pair, 0)


def _make_pass1(n_chunks):
  @functools.partial(
      pl.kernel,
      out_type=jax.ShapeDtypeStruct((NC, N_PAD, FA), jnp.float32),
      mesh=_mesh,
      compiler_params=_sc_params,
      scratch_types=[
          pltpu.VMEM_SHARED((N_PAD, FA), jnp.float32),
          pltpu.VMEM((K, 128), jnp.int32),
          pltpu.VMEM((K, 128), jnp.int32),
          pltpu.VMEM((K, 128), jnp.int32),
          pltpu.VMEM((K, 128), jnp.int32),
          pltpu.VMEM((CHUNK, FA), jnp.float32),
          pltpu.VMEM((CHUNK, FA), jnp.float32),
          pltpu.VMEM((ZROWS, FA), jnp.float32),
          pltpu.SemaphoreType.DMA,
          pltpu.SemaphoreType.DMA,
      ],
  )
  def sc_pass(src_hbm, dst_hbm, table_hbm, acc_out, acc_sh, sidx0, didx0,
              sidx1, didx1, rows0, rows1, zbuf, sem_g, sem_s):
    c = lax.axis_index("c")
    s = lax.axis_index("s")

    def zb(i, carry):
      zbuf[i, :] = jnp.zeros((16,), jnp.float32)
      return carry

    lax.fori_loop(0, ZROWS, zb, 0)

    base_row = s * ROWS_PER_TILE

    def zc(r, carry):
      pltpu.sync_copy(zbuf, acc_sh.at[pl.ds(base_row + r * ZROWS, ZROWS)])
      return carry

    lax.fori_loop(0, ROWS_PER_TILE // ZROWS, zc, 0)
    plsc.subcore_barrier()

    w = c * NS + s


    def load_idx(g, sidx, didx):
      row0 = (w * n_chunks + g) * K
      pltpu.sync_copy(src_hbm.at[pl.ds(row0, K)], sidx)
      pltpu.sync_copy(dst_hbm.at[pl.ds(row0, K)], didx)

    def fire_gathers(sidx, rows):
      return [
          pltpu.async_copy(table_hbm.at[sidx.at[j]],
                           rows.at[pl.ds(j * 128, 128)], sem_g)
          for j in range(K)
      ]

    def fire_scatters(didx, rows):
      return [
          pltpu.async_copy(rows.at[pl.ds(j * 128, 128)],
                           acc_sh.at[didx.at[j]], sem_s, add=True)
          for j in range(K)
      ]

    _edge_loop(n_chunks, w, load_idx, fire_gathers, fire_scatters,
               (sidx0, didx0, sidx1, didx1, rows0, rows1))
    plsc.subcore_barrier()

    pltpu.sync_copy(acc_sh.at[pl.ds(base_row, ROWS_PER_TILE)],
                    acc_out.at[c, pl.ds(base_row, ROWS_PER_TILE)])

  return sc_pass


def _make_pass2(n_chunks):
  @functools.partial(
      pl.kernel,
      out_type=jax.ShapeDtypeStruct((NC, N_PAD, F), jnp.float32),
      mesh=_mesh,
      compiler_params=_sc_params,
      scratch_types=[
          pltpu.VMEM_SHARED((N_PAD, F), jnp.float32),
          pltpu.VMEM_SHARED((N_PAD, F), jnp.float32),
          pltpu.VMEM((K, 128), jnp.int32),
          pltpu.VMEM((K, 128), jnp.int32),
          pltpu.VMEM((K, 128), jnp.int32),
          pltpu.VMEM((K, 128), jnp.int32),
          pltpu.VMEM((CHUNK, F), jnp.float32),
          pltpu.VMEM((CHUNK, F), jnp.float32),
          pltpu.VMEM((ZROWS, F), jnp.float32),
          pltpu.SemaphoreType.DMA,
          pltpu.SemaphoreType.DMA,
      ],
  )
  def sc_pass(src_hbm, dst_hbm, table_hbm, acc_out, acc_sh, table_sp,
              sidx0, didx0, sidx1, didx1, rows0, rows1, zbuf, sem_g, sem_s):
    c = lax.axis_index("c")
    s = lax.axis_index("s")

    def zb(i, carry):
      zbuf[i, :] = jnp.zeros((16,), jnp.float32)[:F]
      return carry

    lax.fori_loop(0, ZROWS, zb, 0)

    base_row = s * ROWS_PER_TILE
    def stage(r, carry):
      row = base_row + r * ZROWS
      pltpu.sync_copy(table_hbm.at[pl.ds(row, ZROWS)], rows0.at[pl.ds(0, ZROWS)])
      pltpu.sync_copy(rows0.at[pl.ds(0, ZROWS)], table_sp.at[pl.ds(row, ZROWS)])
      pltpu.sync_copy(zbuf, acc_sh.at[pl.ds(row, ZROWS)])
      return carry

    lax.fori_loop(0, ROWS_PER_TILE // ZROWS, stage, 0)
    plsc.subcore_barrier()

    w = c * NS + s

    def load_idx(g, sidx, didx):
      row0 = (w * n_chunks + g) * K
      pltpu.sync_copy(src_hbm.at[pl.ds(row0, K)], sidx)
      pltpu.sync_copy(dst_hbm.at[pl.ds(row0, K)], didx)

    def fire_gathers(sidx, rows):
      return [
          pltpu.async_copy(table_sp.at[sidx.at[j]],
                           rows.at[pl.ds(j * 128, 128)], sem_g)
          for j in range(K)
      ]

    def fire_scatters(didx, rows):
      return [
          pltpu.async_copy(rows.at[pl.ds(j * 128, 128)],
                           acc_sh.at[didx.at[j]], sem_s, add=True)
          for j in range(K)
      ]

    _edge_loop(n_chunks, w, load_idx, fire_gathers, fire_scatters,
               (sidx0, didx0, sidx1, didx1, rows0, rows1))
    plsc.subcore_barrier()

    pltpu.sync_copy(acc_sh.at[pl.ds(base_row, ROWS_PER_TILE)],
                    acc_out.at[c, pl.ds(base_row, ROWS_PER_TILE)])

  return sc_pass


_BLK1 = 2048


def _tc_mean(acc):
  def body(acc_ref, out_ref):
    a = acc_ref[0] + acc_ref[1]
    cnt = a[:, 8:9]
    out_ref[...] = a[:, 0:8] / jnp.maximum(cnt, 1.0)

  return pl.pallas_call(
      body,
      grid=(N_PAD // _BLK1,),
      in_specs=[pl.BlockSpec((NC, _BLK1, FA), lambda i: (0, i, 0))],
      out_specs=pl.BlockSpec((_BLK1, F), lambda i: (i, 0)),
      out_shape=jax.ShapeDtypeStruct((N_PAD, F), jnp.float32),
  )(acc)


_BLK2 = 2000


def _tc_final(x, m1, acc1, acc2, w1l, w1r, b1, w2l, w2r, b2):
  def body(x_ref, m1_ref, acc1_ref, acc2_ref, w1l_ref, w1r_ref, b1_ref,
           w2l_ref, w2r_ref, b2_ref, out_ref):
    cnt = (acc1_ref[0] + acc1_ref[1])[:, 8:9]
    m2 = (acc2_ref[0] + acc2_ref[1]) / jnp.maximum(cnt, 1.0)
    mask = (cnt > 0.0).astype(jnp.float32)
    m1b = m1_ref[...]
    dot = functools.partial(jnp.dot, preferred_element_type=jnp.float32)
    h = dot(m1b, w1l_ref[...]) + b1_ref[...] + dot(x_ref[...], w1r_ref[...])
    ah = dot(m2, w1l_ref[...]) + mask * b1_ref[...] + dot(m1b, w1r_ref[...])
    out_ref[...] = dot(ah, w2l_ref[...]) + b2_ref[...] + dot(h, w2r_ref[...])

  return pl.pallas_call(
      body,
      grid=(N_NODES // _BLK2,),
      in_specs=[
          pl.BlockSpec((_BLK2, 8), lambda i: (i, 0)),
          pl.BlockSpec((_BLK2, F), lambda i: (i, 0)),
          pl.BlockSpec((NC, _BLK2, FA), lambda i: (0, i, 0)),
          pl.BlockSpec((NC, _BLK2, F), lambda i: (0, i, 0)),
          pl.BlockSpec((8, 16), lambda i: (0, 0)),
          pl.BlockSpec((8, 16), lambda i: (0, 0)),
          pl.BlockSpec((1, 16), lambda i: (0, 0)),
          pl.BlockSpec((16, 8), lambda i: (0, 0)),
          pl.BlockSpec((16, 8), lambda i: (0, 0)),
          pl.BlockSpec((1, 8), lambda i: (0, 0)),
      ],
      out_specs=pl.BlockSpec((_BLK2, 8), lambda i: (i, 0)),
      out_shape=jax.ShapeDtypeStruct((N_NODES, 8), jnp.float32),
  )(x, m1, acc1, acc2, w1l, w1r, b1, w2l, w2r, b2)


def kernel(x, edge_index, W1_l, W1_r, b1, W2_l, W2_r, b2):
  ei = edge_index.astype(jnp.int32)
  src, dst = ei[0], ei[1]
  e = src.shape[0]
  n_chunks = 2 * -(-e // (NW * CHUNK * 2))
  e_pad = NW * CHUNK * n_chunks
  pad = e_pad - e
  ar = jnp.arange(pad, dtype=jnp.int32)
  src_p = jnp.concatenate([src, ar % N_NODES]).reshape(e_pad // 128, 128)
  dst_p = jnp.concatenate(
      [dst, N_NODES + ar % (N_PAD - N_NODES)]).reshape(e_pad // 128, 128)

  n = x.shape[0]
  x_aug = jnp.concatenate(
      [x, jnp.ones((n, 1), x.dtype), jnp.zeros((n, 7), x.dtype)], axis=1)

  acc1 = _make_pass1(n_chunks)(src_p, dst_p, x_aug)
  m1 = _tc_mean(acc1)
  acc2 = _make_pass2(n_chunks)(src_p, dst_p, m1)

  return _tc_final(x, m1, acc1, acc2, W1_l, W1_r, b1.reshape(1, 16), W2_l,
                   W2_r, b2.reshape(1, 8))


# speedup vs baseline: 1.3751x; 1.0327x over previous
import functools

import jax
import jax.numpy as jnp
from jax import lax
from jax.experimental import pallas as pl
from jax.experimental.pallas import tpu as pltpu
from jax.experimental.pallas import tpu_sc as plsc

N_NODES = 100000
NC, NS = 2, 16
NW = NC * NS
K1 = 6
K2 = 8
ZROWS = 112
ROWS_PER_TILE = 56 * ZROWS
N_PAD = NS * ROWS_PER_TILE
FA = 16
F = 8

_mesh = plsc.VectorSubcoreMesh(
    core_axis_name="c", subcore_axis_name="s", num_cores=NC, num_subcores=NS)

_sc_params = pltpu.CompilerParams(use_tc_tiling_on_sc=False)


def _edge_loop(n_chunks, w, load_idx, fire_gathers, fire_scatters, bufs):
  sidx0, didx0, sidx1, didx1, rows0, rows1 = bufs

  def pair(i, carry):
    load_idx(2 * i, sidx0, didx0)
    ga = fire_gathers(sidx0, rows0)
    load_idx(2 * i + 1, sidx1, didx1)
    for cp in ga:
      cp.wait()
    sa = fire_scatters(didx0, rows0)
    gb = fire_gathers(sidx1, rows1)
    for cp in gb:
      cp.wait()
    sb = fire_scatters(didx1, rows1)
    for cp in sa:
      cp.wait()
    for cp in sb:
      cp.wait()
    return carry

  lax.fori_loop(0, n_chunks // 2, pair, 0)


def _make_pass1(n_chunks, K=K1):
  @functools.partial(
      pl.kernel,
      out_type=jax.ShapeDtypeStruct((NC, N_PAD, FA), jnp.float32),
      mesh=_mesh,
      compiler_params=_sc_params,
      scratch_types=[
          pltpu.VMEM_SHARED((N_PAD, FA), jnp.float32),
          pltpu.VMEM((K, 128), jnp.int32),
          pltpu.VMEM((K, 128), jnp.int32),
          pltpu.VMEM((K, 128), jnp.int32),
          pltpu.VMEM((K, 128), jnp.int32),
          pltpu.VMEM((K * 128, FA), jnp.float32),
          pltpu.VMEM((K * 128, FA), jnp.float32),
          pltpu.VMEM((ZROWS, FA), jnp.float32),
          pltpu.SemaphoreType.DMA,
          pltpu.SemaphoreType.DMA,
      ],
  )
  def sc_pass(src_hbm, dst_hbm, table_hbm, acc_out, acc_sh, sidx0, didx0,
              sidx1, didx1, rows0, rows1, zbuf, sem_g, sem_s):
    c = lax.axis_index("c")
    s = lax.axis_index("s")

    def zb(i, carry):
      zbuf[i, :] = jnp.zeros((16,), jnp.float32)
      return carry

    lax.fori_loop(0, ZROWS, zb, 0)

    base_row = s * ROWS_PER_TILE

    def zc(r, carry):
      pltpu.sync_copy(zbuf, acc_sh.at[pl.ds(base_row + r * ZROWS, ZROWS)])
      return carry

    lax.fori_loop(0, ROWS_PER_TILE // ZROWS, zc, 0)
    plsc.subcore_barrier()

    w = c * NS + s


    def load_idx(g, sidx, didx):
      row0 = (w * n_chunks + g) * K
      pltpu.sync_copy(src_hbm.at[pl.ds(row0, K)], sidx)
      pltpu.sync_copy(dst_hbm.at[pl.ds(row0, K)], didx)

    def fire_gathers(sidx, rows):
      return [
          pltpu.async_copy(table_hbm.at[sidx.at[j]],
                           rows.at[pl.ds(j * 128, 128)], sem_g)
          for j in range(K)
      ]

    def fire_scatters(didx, rows):
      return [
          pltpu.async_copy(rows.at[pl.ds(j * 128, 128)],
                           acc_sh.at[didx.at[j]], sem_s, add=True)
          for j in range(K)
      ]

    _edge_loop(n_chunks, w, load_idx, fire_gathers, fire_scatters,
               (sidx0, didx0, sidx1, didx1, rows0, rows1))
    plsc.subcore_barrier()

    pltpu.sync_copy(acc_sh.at[pl.ds(base_row, ROWS_PER_TILE)],
                    acc_out.at[c, pl.ds(base_row, ROWS_PER_TILE)])

  return sc_pass


def _make_pass2(n_chunks, K=K2):
  @functools.partial(
      pl.kernel,
      out_type=jax.ShapeDtypeStruct((NC, N_PAD, F), jnp.float32),
      mesh=_mesh,
      compiler_params=_sc_params,
      scratch_types=[
          pltpu.VMEM_SHARED((N_PAD, F), jnp.float32),
          pltpu.VMEM_SHARED((N_PAD, F), jnp.float32),
          pltpu.VMEM((K, 128), jnp.int32),
          pltpu.VMEM((K, 128), jnp.int32),
          pltpu.VMEM((K, 128), jnp.int32),
          pltpu.VMEM((K, 128), jnp.int32),
          pltpu.VMEM((K * 128, F), jnp.float32),
          pltpu.VMEM((K * 128, F), jnp.float32),
          pltpu.VMEM((ZROWS, F), jnp.float32),
          pltpu.SemaphoreType.DMA,
          pltpu.SemaphoreType.DMA,
      ],
  )
  def sc_pass(src_hbm, dst_hbm, table_hbm, acc_out, acc_sh, table_sp,
              sidx0, didx0, sidx1, didx1, rows0, rows1, zbuf, sem_g, sem_s):
    c = lax.axis_index("c")
    s = lax.axis_index("s")

    def zb(i, carry):
      zbuf[i, :] = jnp.zeros((16,), jnp.float32)[:F]
      return carry

    lax.fori_loop(0, ZROWS, zb, 0)

    base_row = s * ROWS_PER_TILE
    def stage(r, carry):
      row = base_row + r * ZROWS
      pltpu.sync_copy(table_hbm.at[pl.ds(row, ZROWS)], rows0.at[pl.ds(0, ZROWS)])
      pltpu.sync_copy(rows0.at[pl.ds(0, ZROWS)], table_sp.at[pl.ds(row, ZROWS)])
      pltpu.sync_copy(zbuf, acc_sh.at[pl.ds(row, ZROWS)])
      return carry

    lax.fori_loop(0, ROWS_PER_TILE // ZROWS, stage, 0)
    plsc.subcore_barrier()

    w = c * NS + s

    def load_idx(g, sidx, didx):
      row0 = (w * n_chunks + g) * K
      pltpu.sync_copy(src_hbm.at[pl.ds(row0, K)], sidx)
      pltpu.sync_copy(dst_hbm.at[pl.ds(row0, K)], didx)

    def fire_gathers(sidx, rows):
      return [
          pltpu.async_copy(table_sp.at[sidx.at[j]],
                           rows.at[pl.ds(j * 128, 128)], sem_g)
          for j in range(K)
      ]

    def fire_scatters(didx, rows):
      return [
          pltpu.async_copy(rows.at[pl.ds(j * 128, 128)],
                           acc_sh.at[didx.at[j]], sem_s, add=True)
          for j in range(K)
      ]

    _edge_loop(n_chunks, w, load_idx, fire_gathers, fire_scatters,
               (sidx0, didx0, sidx1, didx1, rows0, rows1))
    plsc.subcore_barrier()

    pltpu.sync_copy(acc_sh.at[pl.ds(base_row, ROWS_PER_TILE)],
                    acc_out.at[c, pl.ds(base_row, ROWS_PER_TILE)])

  return sc_pass


_BLK1 = 2048


def _tc_mean(acc):
  def body(acc_ref, out_ref):
    a = acc_ref[0] + acc_ref[1]
    cnt = a[:, 8:9]
    out_ref[...] = a[:, 0:8] / jnp.maximum(cnt, 1.0)

  return pl.pallas_call(
      body,
      grid=(N_PAD // _BLK1,),
      in_specs=[pl.BlockSpec((NC, _BLK1, FA), lambda i: (0, i, 0))],
      out_specs=pl.BlockSpec((_BLK1, F), lambda i: (i, 0)),
      out_shape=jax.ShapeDtypeStruct((N_PAD, F), jnp.float32),
  )(acc)


_BLK2 = 2000


def _tc_final(x, m1, acc1, acc2, w1l, w1r, b1, w2l, w2r, b2):
  def body(x_ref, m1_ref, acc1_ref, acc2_ref, w1l_ref, w1r_ref, b1_ref,
           w2l_ref, w2r_ref, b2_ref, out_ref):
    cnt = (acc1_ref[0] + acc1_ref[1])[:, 8:9]
    m2 = (acc2_ref[0] + acc2_ref[1]) / jnp.maximum(cnt, 1.0)
    mask = (cnt > 0.0).astype(jnp.float32)
    m1b = m1_ref[...]
    dot = functools.partial(jnp.dot, preferred_element_type=jnp.float32)
    h = dot(m1b, w1l_ref[...]) + b1_ref[...] + dot(x_ref[...], w1r_ref[...])
    ah = dot(m2, w1l_ref[...]) + mask * b1_ref[...] + dot(m1b, w1r_ref[...])
    out_ref[...] = dot(ah, w2l_ref[...]) + b2_ref[...] + dot(h, w2r_ref[...])

  return pl.pallas_call(
      body,
      grid=(N_NODES // _BLK2,),
      in_specs=[
          pl.BlockSpec((_BLK2, 8), lambda i: (i, 0)),
          pl.BlockSpec((_BLK2, F), lambda i: (i, 0)),
          pl.BlockSpec((NC, _BLK2, FA), lambda i: (0, i, 0)),
          pl.BlockSpec((NC, _BLK2, F), lambda i: (0, i, 0)),
          pl.BlockSpec((8, 16), lambda i: (0, 0)),
          pl.BlockSpec((8, 16), lambda i: (0, 0)),
          pl.BlockSpec((1, 16), lambda i: (0, 0)),
          pl.BlockSpec((16, 8), lambda i: (0, 0)),
          pl.BlockSpec((16, 8), lambda i: (0, 0)),
          pl.BlockSpec((1, 8), lambda i: (0, 0)),
      ],
      out_specs=pl.BlockSpec((_BLK2, 8), lambda i: (i, 0)),
      out_shape=jax.ShapeDtypeStruct((N_NODES, 8), jnp.float32),
  )(x, m1, acc1, acc2, w1l, w1r, b1, w2l, w2r, b2)


def kernel(x, edge_index, W1_l, W1_r, b1, W2_l, W2_r, b2):
  ei = edge_index.astype(jnp.int32)
  src, dst = ei[0], ei[1]
  e = src.shape[0]
  grain1, grain2 = NW * K1 * 128 * 2, NW * K2 * 128 * 2
  import math
  grain = math.lcm(grain1, grain2)
  e_pad = grain * -(-e // grain)
  n1 = e_pad // (NW * K1 * 128)
  n2 = e_pad // (NW * K2 * 128)
  pad = e_pad - e
  ar = jnp.arange(pad, dtype=jnp.int32)
  src_p = jnp.concatenate([src, ar % N_NODES]).reshape(e_pad // 128, 128)
  dst_p = jnp.concatenate(
      [dst, N_NODES + ar % (N_PAD - N_NODES)]).reshape(e_pad // 128, 128)

  n = x.shape[0]
  x_aug = jnp.concatenate(
      [x, jnp.ones((n, 1), x.dtype), jnp.zeros((n, 7), x.dtype)], axis=1)

  acc1 = _make_pass1(n1)(src_p, dst_p, x_aug)
  m1 = _tc_mean(acc1)
  acc2 = _make_pass2(n2)(src_p, dst_p, m1)

  return _tc_final(x, m1, acc1, acc2, W1_l, W1_r, b1.reshape(1, 16), W2_l,
                   W2_r, b2.reshape(1, 8))


# speedup vs baseline: 1.3754x; 1.0002x over previous
import functools
import math

import jax
import jax.numpy as jnp
from jax import lax
from jax.experimental import pallas as pl
from jax.experimental.pallas import tpu as pltpu
from jax.experimental.pallas import tpu_sc as plsc

N_NODES = 100000
NC, NS = 2, 16
NW = NC * NS
K1 = 6
K2 = 8
ZROWS = 112
ROWS_PER_TILE = 56 * ZROWS
N_PAD = NS * ROWS_PER_TILE
FA = 16
F = 8

_mesh = plsc.VectorSubcoreMesh(
    core_axis_name="c", subcore_axis_name="s", num_cores=NC, num_subcores=NS)

_sc_params = pltpu.CompilerParams(use_tc_tiling_on_sc=False)


def _edge_loop(n_chunks, load_idx, fire_gathers, fire_scatters, bufs):
  sidx0, didx0, sidx1, didx1, rows0, rows1 = bufs

  def pair(i, carry):
    load_idx(2 * i, sidx0, didx0)
    ga = fire_gathers(sidx0, rows0)
    load_idx(2 * i + 1, sidx1, didx1)
    for cp in ga:
      cp.wait()
    sa = fire_scatters(didx0, rows0)
    gb = fire_gathers(sidx1, rows1)
    for cp in gb:
      cp.wait()
    sb = fire_scatters(didx1, rows1)
    for cp in sa:
      cp.wait()
    for cp in sb:
      cp.wait()
    return carry

  lax.fori_loop(0, n_chunks // 2, pair, 0)


def _make_pass1(n_chunks, K=K1):
  @functools.partial(
      pl.kernel,
      out_type=jax.ShapeDtypeStruct((NC, N_PAD, FA), jnp.float32),
      mesh=_mesh,
      compiler_params=_sc_params,
      scratch_types=[
          pltpu.VMEM_SHARED((N_PAD, FA), jnp.float32),
          pltpu.VMEM((K, 128), jnp.int32),
          pltpu.VMEM((K, 128), jnp.int32),
          pltpu.VMEM((K, 128), jnp.int32),
          pltpu.VMEM((K, 128), jnp.int32),
          pltpu.VMEM((K * 128, FA), jnp.float32),
          pltpu.VMEM((K * 128, FA), jnp.float32),
          pltpu.VMEM((ZROWS, FA), jnp.float32),
          pltpu.SemaphoreType.DMA,
          pltpu.SemaphoreType.DMA,
      ],
  )
  def sc_pass(src_hbm, dst_hbm, table_hbm, acc_out, acc_sh, sidx0, didx0,
              sidx1, didx1, rows0, rows1, zbuf, sem_g, sem_s):
    c = lax.axis_index("c")
    s = lax.axis_index("s")

    def zb(i, carry):
      zbuf[i, :] = jnp.zeros((16,), jnp.float32)
      return carry

    lax.fori_loop(0, ZROWS, zb, 0)

    base_row = s * ROWS_PER_TILE

    def zc(r, carry):
      pltpu.sync_copy(zbuf, acc_sh.at[pl.ds(base_row + r * ZROWS, ZROWS)])
      return carry

    lax.fori_loop(0, ROWS_PER_TILE // ZROWS, zc, 0)
    plsc.subcore_barrier()

    w = c * NS + s


    def load_idx(g, sidx, didx):
      row0 = (w * n_chunks + g) * K
      pltpu.sync_copy(src_hbm.at[pl.ds(row0, K)], sidx)
      pltpu.sync_copy(dst_hbm.at[pl.ds(row0, K)], didx)

    def fire_gathers(sidx, rows):
      return [
          pltpu.async_copy(table_hbm.at[sidx.at[j]],
                           rows.at[pl.ds(j * 128, 128)], sem_g)
          for j in range(K)
      ]

    def fire_scatters(didx, rows):
      return [
          pltpu.async_copy(rows.at[pl.ds(j * 128, 128)],
                           acc_sh.at[didx.at[j]], sem_s, add=True)
          for j in range(K)
      ]

    _edge_loop(n_chunks, load_idx, fire_gathers, fire_scatters,
               (sidx0, didx0, sidx1, didx1, rows0, rows1))
    plsc.subcore_barrier()

    pltpu.sync_copy(acc_sh.at[pl.ds(base_row, ROWS_PER_TILE)],
                    acc_out.at[c, pl.ds(base_row, ROWS_PER_TILE)])

  return sc_pass


def _make_pass2(n_chunks, K=K2):
  @functools.partial(
      pl.kernel,
      out_type=jax.ShapeDtypeStruct((NC, N_PAD, F), jnp.float32),
      mesh=_mesh,
      compiler_params=_sc_params,
      scratch_types=[
          pltpu.VMEM_SHARED((N_PAD, F), jnp.float32),
          pltpu.VMEM_SHARED((N_PAD, F), jnp.float32),
          pltpu.VMEM((K, 128), jnp.int32),
          pltpu.VMEM((K, 128), jnp.int32),
          pltpu.VMEM((K, 128), jnp.int32),
          pltpu.VMEM((K, 128), jnp.int32),
          pltpu.VMEM((K * 128, F), jnp.float32),
          pltpu.VMEM((K * 128, F), jnp.float32),
          pltpu.VMEM((ZROWS, F), jnp.float32),
          pltpu.SemaphoreType.DMA,
          pltpu.SemaphoreType.DMA,
      ],
  )
  def sc_pass(src_hbm, dst_hbm, table_hbm, acc_out, acc_sh, table_sp,
              sidx0, didx0, sidx1, didx1, rows0, rows1, zbuf, sem_g, sem_s):
    c = lax.axis_index("c")
    s = lax.axis_index("s")

    def zb(i, carry):
      zbuf[i, :] = jnp.zeros((16,), jnp.float32)[:F]
      return carry

    lax.fori_loop(0, ZROWS, zb, 0)

    base_row = s * ROWS_PER_TILE
    def stage(r, carry):
      row = base_row + r * ZROWS
      pltpu.sync_copy(table_hbm.at[pl.ds(row, ZROWS)], rows0.at[pl.ds(0, ZROWS)])
      pltpu.sync_copy(rows0.at[pl.ds(0, ZROWS)], table_sp.at[pl.ds(row, ZROWS)])
      pltpu.sync_copy(zbuf, acc_sh.at[pl.ds(row, ZROWS)])
      return carry

    lax.fori_loop(0, ROWS_PER_TILE // ZROWS, stage, 0)
    plsc.subcore_barrier()

    w = c * NS + s

    def load_idx(g, sidx, didx):
      row0 = (w * n_chunks + g) * K
      pltpu.sync_copy(src_hbm.at[pl.ds(row0, K)], sidx)
      pltpu.sync_copy(dst_hbm.at[pl.ds(row0, K)], didx)

    def fire_gathers(sidx, rows):
      return [
          pltpu.async_copy(table_sp.at[sidx.at[j]],
                           rows.at[pl.ds(j * 128, 128)], sem_g)
          for j in range(K)
      ]

    def fire_scatters(didx, rows):
      return [
          pltpu.async_copy(rows.at[pl.ds(j * 128, 128)],
                           acc_sh.at[didx.at[j]], sem_s, add=True)
          for j in range(K)
      ]

    _edge_loop(n_chunks, load_idx, fire_gathers, fire_scatters,
               (sidx0, didx0, sidx1, didx1, rows0, rows1))
    plsc.subcore_barrier()

    pltpu.sync_copy(acc_sh.at[pl.ds(base_row, ROWS_PER_TILE)],
                    acc_out.at[c, pl.ds(base_row, ROWS_PER_TILE)])

  return sc_pass


_BLK1 = 2048


def _tc_mean(acc):
  def body(acc_ref, out_ref):
    a = acc_ref[0] + acc_ref[1]
    cnt = a[:, 8:9]
    out_ref[...] = a[:, 0:8] / jnp.maximum(cnt, 1.0)

  return pl.pallas_call(
      body,
      grid=(N_PAD // _BLK1,),
      in_specs=[pl.BlockSpec((NC, _BLK1, FA), lambda i: (0, i, 0))],
      out_specs=pl.BlockSpec((_BLK1, F), lambda i: (i, 0)),
      out_shape=jax.ShapeDtypeStruct((N_PAD, F), jnp.float32),
  )(acc)


_BLK2 = 2000


def _tc_final(x, m1, acc1, acc2, w1l, w1r, b1, w2l, w2r, b2):
  def body(x_ref, m1_ref, acc1_ref, acc2_ref, w1l_ref, w1r_ref, b1_ref,
           w2l_ref, w2r_ref, b2_ref, out_ref):
    cnt = (acc1_ref[0] + acc1_ref[1])[:, 8:9]
    m2 = (acc2_ref[0] + acc2_ref[1]) / jnp.maximum(cnt, 1.0)
    mask = (cnt > 0.0).astype(jnp.float32)
    m1b = m1_ref[...]
    dot = functools.partial(jnp.dot, preferred_element_type=jnp.float32)
    h = dot(m1b, w1l_ref[...]) + b1_ref[...] + dot(x_ref[...], w1r_ref[...])
    ah = dot(m2, w1l_ref[...]) + mask * b1_ref[...] + dot(m1b, w1r_ref[...])
    out_ref[...] = dot(ah, w2l_ref[...]) + b2_ref[...] + dot(h, w2r_ref[...])

  return pl.pallas_call(
      body,
      grid=(N_NODES // _BLK2,),
      in_specs=[
          pl.BlockSpec((_BLK2, 8), lambda i: (i, 0)),
          pl.BlockSpec((_BLK2, F), lambda i: (i, 0)),
          pl.BlockSpec((NC, _BLK2, FA), lambda i: (0, i, 0)),
          pl.BlockSpec((NC, _BLK2, F), lambda i: (0, i, 0)),
          pl.BlockSpec((8, 16), lambda i: (0, 0)),
          pl.BlockSpec((8, 16), lambda i: (0, 0)),
          pl.BlockSpec((1, 16), lambda i: (0, 0)),
          pl.BlockSpec((16, 8), lambda i: (0, 0)),
          pl.BlockSpec((16, 8), lambda i: (0, 0)),
          pl.BlockSpec((1, 8), lambda i: (0, 0)),
      ],
      out_specs=pl.BlockSpec((_BLK2, 8), lambda i: (i, 0)),
      out_shape=jax.ShapeDtypeStruct((N_NODES, 8), jnp.float32),
  )(x, m1, acc1, acc2, w1l, w1r, b1, w2l, w2r, b2)


def kernel(x, edge_index, W1_l, W1_r, b1, W2_l, W2_r, b2):
  ei = edge_index.astype(jnp.int32)
  src, dst = ei[0], ei[1]
  e = src.shape[0]
  grain1, grain2 = NW * K1 * 128 * 2, NW * K2 * 128 * 2
  grain = math.lcm(grain1, grain2)
  e_pad = grain * -(-e // grain)
  n1 = e_pad // (NW * K1 * 128)
  n2 = e_pad // (NW * K2 * 128)
  pad = e_pad - e
  ar = jnp.arange(pad, dtype=jnp.int32)
  src_p = jnp.concatenate([src, ar % N_NODES]).reshape(e_pad // 128, 128)
  dst_p = jnp.concatenate(
      [dst, N_NODES + ar % (N_PAD - N_NODES)]).reshape(e_pad // 128, 128)

  n = x.shape[0]
  x_aug = jnp.concatenate(
      [x, jnp.ones((n, 1), x.dtype), jnp.zeros((n, 7), x.dtype)], axis=1)

  acc1 = _make_pass1(n1)(src_p, dst_p, x_aug)
  m1 = _tc_mean(acc1)
  acc2 = _make_pass2(n2)(src_p, dst_p, m1)

  return _tc_final(x, m1, acc1, acc2, W1_l, W1_r, b1.reshape(1, 16), W2_l,
                   W2_r, b2.reshape(1, 8))
